# Initial kernel scaffold; baseline (speedup 1.0000x reference)
import jax
import jax.numpy as jnp
from jax.experimental import pallas as pl


def kernel(z, codebook):
    raise NotImplementedError("write your pallas kernel here")


# speedup vs baseline: 1.5124x; 1.5124x over previous
import functools

import jax
import jax.numpy as jnp
from jax.experimental import pallas as pl
from jax.experimental.pallas import tpu as pltpu
from jax.experimental.pallas import tpu_sc as plsc

_NUM_CODES = 8192
_CODE_DIM = 32
_BETA = 0.25
_N_TOKENS = 8192

_TM = 256
_GATHER_WINDOW = 128
_PAD_DIM = 128
_LOSS_SCALE = (1.0 + _BETA) / (_N_TOKENS * _CODE_DIM)


def _argmin_body(a2_ref, z_ref, cbt_ref, b2_ref, idx_ref, loss_ref, acc_ref):
    i = pl.program_id(0)
    dot = jax.lax.dot_general(
        z_ref[...].astype(jnp.bfloat16), cbt_ref[...].astype(jnp.bfloat16),
        (((1,), (0,)), ((), ())),
        preferred_element_type=jnp.float32)
    s = (a2_ref[...] + b2_ref[...]) - 2.0 * dot
    idx_ref[...] = jnp.argmin(s, axis=1).astype(jnp.int32)
    mins = jnp.maximum(jnp.min(s, axis=1), 0.0)

    @pl.when(i == 0)
    def _():
        acc_ref[0] = 0.0

    acc_ref[0] += jnp.sum(mins)

    @pl.when(i == pl.num_programs(0) - 1)
    def _():
        loss_ref[0] = acc_ref[0] * _LOSS_SCALE


def _argmin_call(a2, z, cbt, b2, interpret=False):
    nt = _N_TOKENS // _TM
    return pl.pallas_call(
        _argmin_body,
        grid=(nt,),
        in_specs=[
            pl.BlockSpec((_TM, 1), lambda i: (i, 0)),
            pl.BlockSpec((_TM, _CODE_DIM), lambda i: (i, 0)),
            pl.BlockSpec((_CODE_DIM, _NUM_CODES), lambda i: (0, 0)),
            pl.BlockSpec((1, _NUM_CODES), lambda i: (0, 0)),
        ],
        out_specs=[
            pl.BlockSpec((_TM,), lambda i: (i,)),
            pl.BlockSpec((1,), lambda i: (0,), memory_space=pltpu.SMEM),
        ],
        out_shape=[
            jax.ShapeDtypeStruct((_N_TOKENS,), jnp.int32),
            jax.ShapeDtypeStruct((1,), jnp.float32),
        ],
        scratch_shapes=[pltpu.SMEM((1,), jnp.float32)],
        interpret=interpret,
    )(a2, z, cbt, b2)


def _sc_gather(cb_pad, idx2d):
    mesh = plsc.VectorSubcoreMesh(core_axis_name="core",
                                  subcore_axis_name="subcore")

    @functools.partial(
        pl.kernel,
        out_type=jax.ShapeDtypeStruct((_N_TOKENS, _PAD_DIM), jnp.float32),
        mesh=mesh)
    def gather_kernel(cb_hbm, i_hbm, o_hbm):
        def body(i_vmem, o_vmem):
            pltpu.sync_copy(cb_hbm.at[i_vmem.at[0]], o_vmem)

        pltpu.emit_pipeline(
            body,
            grid=(_N_TOKENS // _GATHER_WINDOW,),
            in_specs=[pl.BlockSpec((1, _GATHER_WINDOW), lambda i: (0, i))],
            out_specs=[pl.BlockSpec((_GATHER_WINDOW, _PAD_DIM),
                                    lambda i: (i, 0))],
            core_axis_name=("core", "subcore"),
            dimension_semantics=(pltpu.PARALLEL,),
        )(i_hbm, o_hbm)

    return gather_kernel(cb_pad, idx2d)


def kernel(z, codebook):
    a2 = jnp.sum(z * z, axis=1, keepdims=True)
    b2 = jnp.sum(codebook * codebook, axis=1, keepdims=True).T
    cbt = codebook.T
    indices, loss = _argmin_call(a2, z, cbt, b2)
    cb_pad = jnp.pad(codebook, ((0, 0), (0, _PAD_DIM - _CODE_DIM)))
    zq_pad = _sc_gather(cb_pad, indices.reshape(1, _N_TOKENS))
    z_q = zq_pad[:, :_CODE_DIM]
    return (z_q, loss[0], indices, z_q)


# speedup vs baseline: 1.6252x; 1.0746x over previous
import functools

import jax
import jax.numpy as jnp
from jax.experimental import pallas as pl
from jax.experimental.pallas import tpu as pltpu
from jax.experimental.pallas import tpu_sc as plsc

_NUM_CODES = 8192
_CODE_DIM = 32
_BETA = 0.25
_N_TOKENS = 8192

_TM = 512
_GATHER_WINDOW = 128
_PAD_DIM = 128
_LOSS_SCALE = (1.0 + _BETA) / (_N_TOKENS * _CODE_DIM)


def _argmin_body(a2_ref, z_ref, cbt_ref, b2_ref, idx_ref, loss_ref, acc_ref):
    i = pl.program_id(0)
    dot = jax.lax.dot_general(
        z_ref[...].astype(jnp.bfloat16), cbt_ref[...].astype(jnp.bfloat16),
        (((1,), (0,)), ((), ())),
        preferred_element_type=jnp.float32)
    s = (a2_ref[...] + b2_ref[...]) - 2.0 * dot
    idx_ref[...] = jnp.argmin(s, axis=1).astype(jnp.int32)
    mins = jnp.maximum(jnp.min(s, axis=1), 0.0)

    @pl.when(i == 0)
    def _():
        acc_ref[0] = 0.0

    acc_ref[0] += jnp.sum(mins)

    @pl.when(i == pl.num_programs(0) - 1)
    def _():
        loss_ref[0] = acc_ref[0] * _LOSS_SCALE


def _argmin_call(a2, z, cbt, b2, interpret=False):
    nt = _N_TOKENS // _TM
    return pl.pallas_call(
        _argmin_body,
        grid=(nt,),
        in_specs=[
            pl.BlockSpec((_TM, 1), lambda i: (i, 0)),
            pl.BlockSpec((_TM, _CODE_DIM), lambda i: (i, 0)),
            pl.BlockSpec((_CODE_DIM, _NUM_CODES), lambda i: (0, 0)),
            pl.BlockSpec((1, _NUM_CODES), lambda i: (0, 0)),
        ],
        out_specs=[
            pl.BlockSpec((_TM,), lambda i: (i,)),
            pl.BlockSpec((1,), lambda i: (0,), memory_space=pltpu.SMEM),
        ],
        out_shape=[
            jax.ShapeDtypeStruct((_N_TOKENS,), jnp.int32),
            jax.ShapeDtypeStruct((1,), jnp.float32),
        ],
        scratch_shapes=[pltpu.SMEM((1,), jnp.float32)],
        interpret=interpret,
    )(a2, z, cbt, b2)


def _sc_gather(cb_pad, idx2d):
    mesh = plsc.VectorSubcoreMesh(core_axis_name="core",
                                  subcore_axis_name="subcore")

    @functools.partial(
        pl.kernel,
        out_type=jax.ShapeDtypeStruct((_N_TOKENS, _PAD_DIM), jnp.float32),
        mesh=mesh)
    def gather_kernel(cb_hbm, i_hbm, o_hbm):
        def body(i_vmem, o_vmem):
            pltpu.sync_copy(cb_hbm.at[i_vmem.at[0]], o_vmem)

        pltpu.emit_pipeline(
            body,
            grid=(_N_TOKENS // _GATHER_WINDOW,),
            in_specs=[pl.BlockSpec((1, _GATHER_WINDOW), lambda i: (0, i))],
            out_specs=[pl.BlockSpec((_GATHER_WINDOW, _PAD_DIM),
                                    lambda i: (i, 0))],
            core_axis_name=("core", "subcore"),
            dimension_semantics=(pltpu.PARALLEL,),
        )(i_hbm, o_hbm)

    return gather_kernel(cb_pad, idx2d)


def kernel(z, codebook):
    a2 = jnp.sum(z * z, axis=1, keepdims=True)
    b2 = jnp.sum(codebook * codebook, axis=1, keepdims=True).T
    cbt = codebook.T
    indices, loss = _argmin_call(a2, z, cbt, b2)
    cb_pad = jnp.pad(codebook, ((0, 0), (0, _PAD_DIM - _CODE_DIM)))
    zq_pad = _sc_gather(cb_pad, indices.reshape(1, _N_TOKENS))
    z_q = zq_pad[:, :_CODE_DIM]
    return (z_q, loss[0], indices, z_q)
